# Initial kernel scaffold; baseline (speedup 1.0000x reference)
import jax
import jax.numpy as jnp
from jax.experimental import pallas as pl


def kernel(x, edge_index, edge_attr, batch, pe, params):
    raise NotImplementedError("write your pallas kernel here")


# speedup vs baseline: 15.5883x; 15.5883x over previous
import functools

import jax
import jax.numpy as jnp
import numpy as np
from jax import lax
from jax.experimental import pallas as pl
from jax.experimental.pallas import tpu as pltpu
from jax.experimental.pallas import tpu_sc as plsc

N = 10000
E = 320000
HID = 128
H = 8
C = 16
B = 16
ACCW = 144

W = 80
NTILES = 32
EPT = E // NTILES
NCH = EPT // W
RPT = N // 16

ROWB = 1000
NROWB = N // ROWB


def _edge_body(q_hbm, kv_hbm, src_hbm, dst_hbm, ea_hbm, wep_hbm, zeros_hbm,
               zerosd_hbm, outm_hbm, outd_hbm, srci, dsti, qbuf, kvbuf, eabuf,
               msgbuf, denbuf, wepbuf, accm, accd):
    c = lax.axis_index("c")
    s = lax.axis_index("s")
    wid = s * 2 + c

    pltpu.sync_copy(zeros_hbm, accm.at[pl.ds(s * RPT, RPT), :])
    pltpu.sync_copy(zerosd_hbm, accd.at[pl.ds(s * RPT, RPT), :])
    pltpu.sync_copy(wep_hbm, wepbuf)
    plsc.subcore_barrier()

    lane = lax.iota(jnp.int32, 16)
    zeros16i = jnp.zeros((16,), jnp.int32)
    ones16i = jnp.ones((16,), jnp.int32)
    lane_lt8 = lane < 8

    @pl.loop(0, NCH)
    def _chunk(ci):
        base = wid * EPT + ci * W
        pltpu.sync_copy(src_hbm.at[pl.ds(base, W)], srci)
        pltpu.sync_copy(dst_hbm.at[pl.ds(base, W)], dsti)
        pltpu.sync_copy(ea_hbm.at[pl.ds(base, W), :], eabuf)
        pltpu.sync_copy(kv_hbm.at[srci], kvbuf)
        pltpu.sync_copy(q_hbm.at[dsti], qbuf)

        @pl.loop(0, W)
        def _edge(w):
            wv = jnp.full((16,), w, jnp.int32)
            ea0 = plsc.load_gather(eabuf, [wv, zeros16i])
            ea1 = plsc.load_gather(eabuf, [wv, ones16i])
            tail = jnp.zeros((16,), jnp.float32)
            for h in range(H):
                sl = pl.ds(h * 16, 16)
                e = ea0 * wepbuf[0, sl] + ea1 * wepbuf[1, sl] + wepbuf[2, sl]
                qh = qbuf[w, sl] * 0.25
                kh = kvbuf[w, sl]
                vh = kvbuf[w, pl.ds(128 + h * 16, 16)]
                a = jnp.sum(qh * (kh + e))
                av = jnp.minimum(jnp.full((16,), a), 60.0)
                expa = jnp.exp(av)
                msgbuf[w, sl] = (vh + e) * expa
                tail = jnp.where(lane == h, expa, tail)
            plsc.store_scatter(denbuf, [wv, lane], tail, mask=lane_lt8)

        pltpu.sync_copy(msgbuf, accm.at[dsti], add=True)
        pltpu.sync_copy(denbuf, accd.at[dsti], add=True)

    plsc.subcore_barrier()
    pltpu.sync_copy(accm.at[pl.ds(s * RPT, RPT), :],
                    outm_hbm.at[c, pl.ds(s * RPT, RPT), :])
    pltpu.sync_copy(accd.at[pl.ds(s * RPT, RPT), :],
                    outd_hbm.at[c, pl.ds(s * RPT, RPT), :])


def _edge_pass(q, kv, src, dst, ea, wepack, zeros, zerosd):
    mesh = plsc.VectorSubcoreMesh(core_axis_name="c", subcore_axis_name="s")
    k = functools.partial(
        pl.kernel,
        mesh=mesh,
        compiler_params=pltpu.CompilerParams(use_tc_tiling_on_sc=False,
                                             needs_layout_passes=False),
        out_type=[
            jax.ShapeDtypeStruct((2, N, HID), jnp.float32),
            jax.ShapeDtypeStruct((2, N, H), jnp.float32),
        ],
        scratch_types=[
            pltpu.VMEM((W,), jnp.int32),
            pltpu.VMEM((W,), jnp.int32),
            pltpu.VMEM((W, HID), jnp.float32),
            pltpu.VMEM((W, 2 * HID), jnp.float32),
            pltpu.VMEM((W, 2), jnp.float32),
            pltpu.VMEM((W, HID), jnp.float32),
            pltpu.VMEM((W, H), jnp.float32),
            pltpu.VMEM((3, HID), jnp.float32),
            pltpu.VMEM_SHARED((N, HID), jnp.float32),
            pltpu.VMEM_SHARED((N, H), jnp.float32),
        ],
    )(_edge_body)
    return k(q, kv, src, dst, ea, wepack, zeros, zerosd)


def _ln(x, g, b):
    m = jnp.mean(x, axis=-1, keepdims=True)
    v = jnp.mean((x - m) ** 2, axis=-1, keepdims=True)
    return (x - m) / jnp.sqrt(v + 1e-5) * g + b


def _embed_body(x_ref, pe_ref, win_ref, bin_ref, g_ref, b_ref, wpe_ref,
                bpe_ref, o_ref):
    h = _ln(x_ref[...] @ win_ref[...] + bin_ref[...], g_ref[...], b_ref[...])
    o_ref[...] = h + pe_ref[...] @ wpe_ref[...] + bpe_ref[...]


def _embed(x, pe, win, bin_, g, b, wpe, bpe):
    full = lambda shp: pl.BlockSpec(shp, lambda i: (0,) * len(shp))
    return pl.pallas_call(
        _embed_body,
        grid=(NROWB,),
        in_specs=[
            pl.BlockSpec((ROWB, HID), lambda i: (i, 0)),
            pl.BlockSpec((ROWB, 8), lambda i: (i, 0)),
            full((HID, HID)), full((1, HID)), full((1, HID)), full((1, HID)),
            full((8, HID)), full((1, HID)),
        ],
        out_specs=pl.BlockSpec((ROWB, HID), lambda i: (i, 0)),
        out_shape=jax.ShapeDtypeStruct((N, HID), jnp.float32),
    )(x, pe, win, bin_, g, b, wpe, bpe)


def _qkv_body(h_ref, w_ref, b_ref, q_ref, kv_ref):
    y = h_ref[...] @ w_ref[...] + b_ref[...]
    q_ref[...] = y[:, :HID]
    kv_ref[...] = y[:, HID:]


def _qkv(h, wqkv, bqkv):
    full = lambda shp: pl.BlockSpec(shp, lambda i: (0,) * len(shp))
    return pl.pallas_call(
        _qkv_body,
        grid=(NROWB,),
        in_specs=[
            pl.BlockSpec((ROWB, HID), lambda i: (i, 0)),
            full((HID, 3 * HID)), full((1, 3 * HID)),
        ],
        out_specs=[
            pl.BlockSpec((ROWB, HID), lambda i: (i, 0)),
            pl.BlockSpec((ROWB, 2 * HID), lambda i: (i, 0)),
        ],
        out_shape=[
            jax.ShapeDtypeStruct((N, HID), jnp.float32),
            jax.ShapeDtypeStruct((N, 2 * HID), jnp.float32),
        ],
    )(h, wqkv, bqkv)


def _post_body(accm_ref, accd_ref, h_ref, e8_ref, ws_ref, bs_ref, g_ref,
               b_ref, w1_ref, b1_ref, w2_ref, b2_ref, o_ref):
    a = accm_ref[0] + accm_ref[1]
    d = accd_ref[0] + accd_ref[1]
    denx = d @ e8_ref[...]
    attn = a / (denx + 1e-16)
    h = h_ref[...]
    h2 = attn + h @ ws_ref[...] + bs_ref[...]
    hh = _ln(h + h2, g_ref[...], b_ref[...])
    f = jnp.maximum(hh @ w1_ref[...] + b1_ref[...], 0.0) @ w2_ref[...] \
        + b2_ref[...]
    o_ref[...] = _ln(hh + f, g_ref[...], b_ref[...])


def _post(accm, accd, h, e8, ws, bs, g, b, w1, b1, w2, b2):
    full = lambda shp: pl.BlockSpec(shp, lambda i: (0,) * len(shp))
    return pl.pallas_call(
        _post_body,
        grid=(NROWB,),
        in_specs=[
            pl.BlockSpec((2, ROWB, HID), lambda i: (0, i, 0)),
            pl.BlockSpec((2, ROWB, H), lambda i: (0, i, 0)),
            pl.BlockSpec((ROWB, HID), lambda i: (i, 0)),
            full((H, HID)), full((HID, HID)), full((1, HID)),
            full((1, HID)), full((1, HID)),
            full((HID, 4 * HID)), full((1, 4 * HID)),
            full((4 * HID, HID)), full((1, HID)),
        ],
        out_specs=pl.BlockSpec((ROWB, HID), lambda i: (i, 0)),
        out_shape=jax.ShapeDtypeStruct((N, HID), jnp.float32),
    )(accm, accd, h, e8, ws, bs, g, b, w1, b1, w2, b2)


def _pool_body(h_ref, bt_ref, wt1_ref, bt1_ref, wt2_ref, bt2_ref, wg1_ref,
               bg1_ref, wg2_ref, bg2_ref, t_ref, g_ref, pacc, cacc):
    i = pl.program_id(0)

    @pl.when(i == 0)
    def _():
        pacc[...] = jnp.zeros((B, HID), jnp.float32)
        cacc[...] = jnp.zeros((B, HID), jnp.float32)

    bb = jnp.broadcast_to(bt_ref[0], (B, ROWB))
    ids = lax.broadcasted_iota(jnp.int32, (B, ROWB), 0).astype(jnp.float32)
    oh = (ids == bb).astype(jnp.float32)
    pacc[...] += oh @ h_ref[...]
    cacc[...] += jnp.broadcast_to(jnp.sum(oh, axis=1, keepdims=True), (B, HID))

    @pl.when(i == NROWB - 1)
    def _():
        pooled = pacc[...] / jnp.maximum(cacc[...], 1.0)
        z = jnp.maximum(pooled @ wt1_ref[...] + bt1_ref[...], 0.0)
        tv = z @ wt2_ref[...] + bt2_ref[...]
        t_ref[...] = jnp.broadcast_to(tv[:, :1], (B, HID))
        zg = jnp.maximum(pooled @ wg1_ref[...] + bg1_ref[...], 0.0)
        gv = jax.nn.sigmoid(zg @ wg2_ref[...] + bg2_ref[...]) * 2.0
        g_ref[...] = jnp.broadcast_to(gv[:, :1], (B, HID))


def _pool(h, batchf, wt1, bt1, wt2, bt2, wg1, bg1, wg2, bg2):
    full = lambda shp: pl.BlockSpec(shp, lambda i: (0,) * len(shp))
    out = pl.pallas_call(
        _pool_body,
        grid=(NROWB,),
        in_specs=[
            pl.BlockSpec((ROWB, HID), lambda i: (i, 0)),
            pl.BlockSpec((1, 1, ROWB), lambda i: (i, 0, 0)),
            full((HID, 64)), full((1, 64)), full((64, HID)), full((1, HID)),
            full((HID, 64)), full((1, 64)), full((64, HID)), full((1, HID)),
        ],
        out_specs=[
            pl.BlockSpec((B, HID), lambda i: (0, 0)),
            pl.BlockSpec((B, HID), lambda i: (0, 0)),
        ],
        out_shape=[
            jax.ShapeDtypeStruct((B, HID), jnp.float32),
            jax.ShapeDtypeStruct((B, HID), jnp.float32),
        ],
        scratch_shapes=[
            pltpu.VMEM((B, HID), jnp.float32),
            pltpu.VMEM((B, HID), jnp.float32),
        ],
    )(h, batchf, wt1, bt1, wt2, bt2, wg1, bg1, wg2, bg2)
    return out


_E8 = np.kron(np.eye(H, dtype=np.float32), np.ones((1, C), np.float32))


def kernel(x, edge_index, edge_attr, batch, pe, params):
    with jax.default_matmul_precision("highest"):
        return _kernel_impl(x, edge_index, edge_attr, batch, pe, params)


def _kernel_impl(x, edge_index, edge_attr, batch, pe, params):
    p = params
    src = edge_index[0].astype(jnp.int32)
    dst = edge_index[1].astype(jnp.int32)
    ea = edge_attr.astype(jnp.float32)
    e8 = jnp.asarray(_E8)
    zeros = jnp.zeros((RPT, HID), jnp.float32)
    zerosd = jnp.zeros((RPT, H), jnp.float32)
    r = lambda v: v.reshape(1, -1)

    h = _embed(x, pe, p['Win'], r(p['bin']), r(p['g_in']), r(p['b_in']),
               p['Wpe'], r(p['bpe']))

    for lp in p['layers']:
        wqkv = jnp.concatenate([lp['Wq'], lp['Wk'], lp['Wv']], axis=1)
        bqkv = jnp.concatenate([lp['bq'], lp['bk'], lp['bv']]).reshape(1, -1)
        wepack = jnp.concatenate([lp['We'], lp['be'].reshape(1, -1)], axis=0)
        q, kv = _qkv(h, wqkv, bqkv)
        accm, accd = _edge_pass(q, kv, src, dst, ea, wepack, zeros, zerosd)
        h = _post(accm, accd, h, e8, lp['Ws'], r(lp['bs']), r(lp['ln_g']),
                  r(lp['ln_b']), lp['W1'], r(lp['b1']), lp['W2'], r(lp['b2']))

    batchf = batch.astype(jnp.float32).reshape(NROWB, 1, ROWB)
    pad2 = lambda w: jnp.pad(w, ((0, 0), (0, HID - w.shape[1])))
    padb = lambda v: jnp.pad(v.reshape(1, -1), ((0, 0), (0, HID - v.shape[0])))
    t2, g2 = _pool(h, batchf, p['Wt1'], r(p['bt1']), pad2(p['Wt2']),
                   padb(p['bt2']), p['Wg1'], r(p['bg1']), pad2(p['Wg2']),
                   padb(p['bg2']))
    return (t2[:, 0], g2[:, 0])


# speedup vs baseline: 38.1582x; 2.4479x over previous
import functools

import jax
import jax.numpy as jnp
import numpy as np
from jax import lax
from jax.experimental import pallas as pl
from jax.experimental.pallas import tpu as pltpu
from jax.experimental.pallas import tpu_sc as plsc

N = 10000
E = 320000
HID = 128
H = 8
C = 16
B = 16
ACCW = 144

W = 80
NTILES = 32
EPT = E // NTILES
NCH = EPT // W
RPT = N // 16

ROWB = 1000
NROWB = N // ROWB


def _edge_body(q_hbm, kv_hbm, src_hbm, dst_hbm, ea_hbm, wep_hbm, zeros_hbm,
               zerosd_hbm, outm_hbm, outd_hbm, srci, dsti, qbuf, kvbuf, eabuf,
               msgbuf, denbuf, wepbuf, accm, accd):
    c = lax.axis_index("c")
    s = lax.axis_index("s")
    wid = s * 2 + c

    pltpu.sync_copy(zeros_hbm, accm.at[pl.ds(s * RPT, RPT), :])
    pltpu.sync_copy(zerosd_hbm, accd.at[pl.ds(s * RPT, RPT), :])
    pltpu.sync_copy(wep_hbm, wepbuf)
    plsc.subcore_barrier()

    lane = lax.iota(jnp.int32, 16)
    zeros16i = jnp.zeros((16,), jnp.int32)
    ones16i = jnp.ones((16,), jnp.int32)
    lane_lt8 = lane < 8
    perms = [(lane ^ 1)[:, None], (lane ^ 2)[:, None], (lane ^ 4)[:, None],
             (lane ^ 8)[:, None]]
    gdn = lax.GatherDimensionNumbers(offset_dims=(), collapsed_slice_dims=(0,),
                                     start_index_map=(0,))
    shuf = lambda t, p: lax.gather(
        t, p, gdn, (1,), mode=lax.GatherScatterMode.PROMISE_IN_BOUNDS)

    @pl.loop(0, NCH)
    def _chunk(ci):
        base = wid * EPT + ci * W
        pltpu.sync_copy(src_hbm.at[pl.ds(base, W)], srci)
        pltpu.sync_copy(dst_hbm.at[pl.ds(base, W)], dsti)
        pltpu.sync_copy(ea_hbm.at[pl.ds(base, W), :], eabuf)
        pltpu.sync_copy(kv_hbm.at[srci], kvbuf)
        pltpu.sync_copy(q_hbm.at[dsti], qbuf)

        we0 = [wepbuf[0, pl.ds(h * 16, 16)] for h in range(H)]
        we1 = [wepbuf[1, pl.ds(h * 16, 16)] for h in range(H)]
        be = [wepbuf[2, pl.ds(h * 16, 16)] for h in range(H)]

        @pl.loop(0, W)
        def _edge(w):
            wv = jnp.full((16,), w, jnp.int32)
            ea0 = plsc.load_gather(eabuf, [wv, zeros16i])
            ea1 = plsc.load_gather(eabuf, [wv, ones16i])
            es, ts, vse = [], [], []
            for h in range(H):
                es.append(ea0 * we0[h] + ea1 * we1[h] + be[h])
            for h in range(H):
                sl = pl.ds(h * 16, 16)
                ts.append(qbuf[w, sl] * (kvbuf[w, sl] + es[h]))
                vse.append(kvbuf[w, pl.ds(128 + h * 16, 16)] + es[h])
            for p in perms:
                ts = [t + shuf(t, p) for t in ts]
            exs = [jnp.exp(jnp.minimum(t, 60.0)) for t in ts]
            for h in range(H):
                msgbuf[w, pl.ds(h * 16, 16)] = vse[h] * exs[h]
            tail = jnp.zeros((16,), jnp.float32)
            for h in range(H):
                tail = jnp.where(lane == h, exs[h], tail)
            plsc.store_scatter(denbuf, [wv, lane], tail, mask=lane_lt8)

        pltpu.sync_copy(msgbuf, accm.at[dsti], add=True)
        pltpu.sync_copy(denbuf, accd.at[dsti], add=True)

    plsc.subcore_barrier()
    pltpu.sync_copy(accm.at[pl.ds(s * RPT, RPT), :],
                    outm_hbm.at[c, pl.ds(s * RPT, RPT), :])
    pltpu.sync_copy(accd.at[pl.ds(s * RPT, RPT), :],
                    outd_hbm.at[c, pl.ds(s * RPT, RPT), :])


def _edge_pass(q, kv, src, dst, ea, wepack, zeros, zerosd):
    mesh = plsc.VectorSubcoreMesh(core_axis_name="c", subcore_axis_name="s")
    k = functools.partial(
        pl.kernel,
        mesh=mesh,
        compiler_params=pltpu.CompilerParams(use_tc_tiling_on_sc=False,
                                             needs_layout_passes=False),
        out_type=[
            jax.ShapeDtypeStruct((2, N, HID), jnp.float32),
            jax.ShapeDtypeStruct((2, N, H), jnp.float32),
        ],
        scratch_types=[
            pltpu.VMEM((W,), jnp.int32),
            pltpu.VMEM((W,), jnp.int32),
            pltpu.VMEM((W, HID), jnp.float32),
            pltpu.VMEM((W, 2 * HID), jnp.float32),
            pltpu.VMEM((W, 2), jnp.float32),
            pltpu.VMEM((W, HID), jnp.float32),
            pltpu.VMEM((W, H), jnp.float32),
            pltpu.VMEM((3, HID), jnp.float32),
            pltpu.VMEM_SHARED((N, HID), jnp.float32),
            pltpu.VMEM_SHARED((N, H), jnp.float32),
        ],
    )(_edge_body)
    return k(q, kv, src, dst, ea, wepack, zeros, zerosd)


def _ln(x, g, b):
    m = jnp.mean(x, axis=-1, keepdims=True)
    v = jnp.mean((x - m) ** 2, axis=-1, keepdims=True)
    return (x - m) / jnp.sqrt(v + 1e-5) * g + b


def _embed_body(x_ref, pe_ref, win_ref, bin_ref, g_ref, b_ref, wpe_ref,
                bpe_ref, o_ref):
    h = _ln(x_ref[...] @ win_ref[...] + bin_ref[...], g_ref[...], b_ref[...])
    o_ref[...] = h + pe_ref[...] @ wpe_ref[...] + bpe_ref[...]


def _embed(x, pe, win, bin_, g, b, wpe, bpe):
    full = lambda shp: pl.BlockSpec(shp, lambda i: (0,) * len(shp))
    return pl.pallas_call(
        _embed_body,
        grid=(NROWB,),
        in_specs=[
            pl.BlockSpec((ROWB, HID), lambda i: (i, 0)),
            pl.BlockSpec((ROWB, 8), lambda i: (i, 0)),
            full((HID, HID)), full((1, HID)), full((1, HID)), full((1, HID)),
            full((8, HID)), full((1, HID)),
        ],
        out_specs=pl.BlockSpec((ROWB, HID), lambda i: (i, 0)),
        out_shape=jax.ShapeDtypeStruct((N, HID), jnp.float32),
    )(x, pe, win, bin_, g, b, wpe, bpe)


def _qkv_body(h_ref, w_ref, b_ref, q_ref, kv_ref):
    y = h_ref[...] @ w_ref[...] + b_ref[...]
    q_ref[...] = y[:, :HID]
    kv_ref[...] = y[:, HID:]


def _qkv(h, wqkv, bqkv):
    full = lambda shp: pl.BlockSpec(shp, lambda i: (0,) * len(shp))
    return pl.pallas_call(
        _qkv_body,
        grid=(NROWB,),
        in_specs=[
            pl.BlockSpec((ROWB, HID), lambda i: (i, 0)),
            full((HID, 3 * HID)), full((1, 3 * HID)),
        ],
        out_specs=[
            pl.BlockSpec((ROWB, HID), lambda i: (i, 0)),
            pl.BlockSpec((ROWB, 2 * HID), lambda i: (i, 0)),
        ],
        out_shape=[
            jax.ShapeDtypeStruct((N, HID), jnp.float32),
            jax.ShapeDtypeStruct((N, 2 * HID), jnp.float32),
        ],
    )(h, wqkv, bqkv)


def _post_body(accm_ref, accd_ref, h_ref, e8_ref, ws_ref, bs_ref, g_ref,
               b_ref, w1_ref, b1_ref, w2_ref, b2_ref, o_ref):
    a = accm_ref[0] + accm_ref[1]
    d = accd_ref[0] + accd_ref[1]
    denx = d @ e8_ref[...]
    attn = a / (denx + 1e-16)
    h = h_ref[...]
    h2 = attn + h @ ws_ref[...] + bs_ref[...]
    hh = _ln(h + h2, g_ref[...], b_ref[...])
    f = jnp.maximum(hh @ w1_ref[...] + b1_ref[...], 0.0) @ w2_ref[...] \
        + b2_ref[...]
    o_ref[...] = _ln(hh + f, g_ref[...], b_ref[...])


def _post(accm, accd, h, e8, ws, bs, g, b, w1, b1, w2, b2):
    full = lambda shp: pl.BlockSpec(shp, lambda i: (0,) * len(shp))
    return pl.pallas_call(
        _post_body,
        grid=(NROWB,),
        in_specs=[
            pl.BlockSpec((2, ROWB, HID), lambda i: (0, i, 0)),
            pl.BlockSpec((2, ROWB, H), lambda i: (0, i, 0)),
            pl.BlockSpec((ROWB, HID), lambda i: (i, 0)),
            full((H, HID)), full((HID, HID)), full((1, HID)),
            full((1, HID)), full((1, HID)),
            full((HID, 4 * HID)), full((1, 4 * HID)),
            full((4 * HID, HID)), full((1, HID)),
        ],
        out_specs=pl.BlockSpec((ROWB, HID), lambda i: (i, 0)),
        out_shape=jax.ShapeDtypeStruct((N, HID), jnp.float32),
    )(accm, accd, h, e8, ws, bs, g, b, w1, b1, w2, b2)


def _pool_body(h_ref, bt_ref, wt1_ref, bt1_ref, wt2_ref, bt2_ref, wg1_ref,
               bg1_ref, wg2_ref, bg2_ref, t_ref, g_ref, pacc, cacc):
    i = pl.program_id(0)

    @pl.when(i == 0)
    def _():
        pacc[...] = jnp.zeros((B, HID), jnp.float32)
        cacc[...] = jnp.zeros((B, HID), jnp.float32)

    bb = jnp.broadcast_to(bt_ref[0], (B, ROWB))
    ids = lax.broadcasted_iota(jnp.int32, (B, ROWB), 0).astype(jnp.float32)
    oh = (ids == bb).astype(jnp.float32)
    pacc[...] += oh @ h_ref[...]
    cacc[...] += jnp.broadcast_to(jnp.sum(oh, axis=1, keepdims=True), (B, HID))

    @pl.when(i == NROWB - 1)
    def _():
        pooled = pacc[...] / jnp.maximum(cacc[...], 1.0)
        z = jnp.maximum(pooled @ wt1_ref[...] + bt1_ref[...], 0.0)
        tv = z @ wt2_ref[...] + bt2_ref[...]
        t_ref[...] = jnp.broadcast_to(tv[:, :1], (B, HID))
        zg = jnp.maximum(pooled @ wg1_ref[...] + bg1_ref[...], 0.0)
        gv = jax.nn.sigmoid(zg @ wg2_ref[...] + bg2_ref[...]) * 2.0
        g_ref[...] = jnp.broadcast_to(gv[:, :1], (B, HID))


def _pool(h, batchf, wt1, bt1, wt2, bt2, wg1, bg1, wg2, bg2):
    full = lambda shp: pl.BlockSpec(shp, lambda i: (0,) * len(shp))
    out = pl.pallas_call(
        _pool_body,
        grid=(NROWB,),
        in_specs=[
            pl.BlockSpec((ROWB, HID), lambda i: (i, 0)),
            pl.BlockSpec((1, 1, ROWB), lambda i: (i, 0, 0)),
            full((HID, 64)), full((1, 64)), full((64, HID)), full((1, HID)),
            full((HID, 64)), full((1, 64)), full((64, HID)), full((1, HID)),
        ],
        out_specs=[
            pl.BlockSpec((B, HID), lambda i: (0, 0)),
            pl.BlockSpec((B, HID), lambda i: (0, 0)),
        ],
        out_shape=[
            jax.ShapeDtypeStruct((B, HID), jnp.float32),
            jax.ShapeDtypeStruct((B, HID), jnp.float32),
        ],
        scratch_shapes=[
            pltpu.VMEM((B, HID), jnp.float32),
            pltpu.VMEM((B, HID), jnp.float32),
        ],
    )(h, batchf, wt1, bt1, wt2, bt2, wg1, bg1, wg2, bg2)
    return out


_E8 = np.kron(np.eye(H, dtype=np.float32), np.ones((1, C), np.float32))


def kernel(x, edge_index, edge_attr, batch, pe, params):
    with jax.default_matmul_precision("highest"):
        return _kernel_impl(x, edge_index, edge_attr, batch, pe, params)


def _kernel_impl(x, edge_index, edge_attr, batch, pe, params):
    p = params
    src = edge_index[0].astype(jnp.int32)
    dst = edge_index[1].astype(jnp.int32)
    ea = edge_attr.astype(jnp.float32)
    e8 = jnp.asarray(_E8)
    zeros = jnp.zeros((RPT, HID), jnp.float32)
    zerosd = jnp.zeros((RPT, H), jnp.float32)
    r = lambda v: v.reshape(1, -1)

    h = _embed(x, pe, p['Win'], r(p['bin']), r(p['g_in']), r(p['b_in']),
               p['Wpe'], r(p['bpe']))

    for lp in p['layers']:
        wqkv = jnp.concatenate([lp['Wq'] * 0.25, lp['Wk'], lp['Wv']], axis=1)
        bqkv = jnp.concatenate([lp['bq'] * 0.25, lp['bk'],
                                lp['bv']]).reshape(1, -1)
        wepack = jnp.concatenate([lp['We'], lp['be'].reshape(1, -1)], axis=0)
        q, kv = _qkv(h, wqkv, bqkv)
        accm, accd = _edge_pass(q, kv, src, dst, ea, wepack, zeros, zerosd)
        h = _post(accm, accd, h, e8, lp['Ws'], r(lp['bs']), r(lp['ln_g']),
                  r(lp['ln_b']), lp['W1'], r(lp['b1']), lp['W2'], r(lp['b2']))

    batchf = batch.astype(jnp.float32).reshape(NROWB, 1, ROWB)
    pad2 = lambda w: jnp.pad(w, ((0, 0), (0, HID - w.shape[1])))
    padb = lambda v: jnp.pad(v.reshape(1, -1), ((0, 0), (0, HID - v.shape[0])))
    t2, g2 = _pool(h, batchf, p['Wt1'], r(p['bt1']), pad2(p['Wt2']),
                   padb(p['bt2']), p['Wg1'], r(p['bg1']), pad2(p['Wg2']),
                   padb(p['bg2']))
    return (t2[:, 0], g2[:, 0])


# speedup vs baseline: 39.3545x; 1.0314x over previous
import functools

import jax
import jax.numpy as jnp
import numpy as np
from jax import lax
from jax.experimental import pallas as pl
from jax.experimental.pallas import tpu as pltpu
from jax.experimental.pallas import tpu_sc as plsc

N = 10000
E = 320000
HID = 128
H = 8
C = 16
B = 16
ACCW = 144

W = 80
NTILES = 32
EPT = E // NTILES
NCH = EPT // W
RPT = N // 16

ROWB = 1000
NROWB = N // ROWB


def _edge_body(q_hbm, kv_hbm, src_hbm, dst_hbm, ea_hbm, wep_hbm, zeros_hbm,
               zerosd_hbm, outm_hbm, outd_hbm, srci, dsti, qbuf, kvbuf, eabuf,
               msgbuf, denbuf, wepbuf, accm, accd):
    c = lax.axis_index("c")
    s = lax.axis_index("s")
    wid = s * 2 + c

    pltpu.sync_copy(zeros_hbm, accm.at[pl.ds(s * RPT, RPT), :])
    pltpu.sync_copy(zerosd_hbm, accd.at[pl.ds(s * RPT, RPT), :])
    pltpu.sync_copy(wep_hbm, wepbuf)
    plsc.subcore_barrier()

    lane = lax.iota(jnp.int32, 16)
    zeros16i = jnp.zeros((16,), jnp.int32)
    ones16i = jnp.ones((16,), jnp.int32)
    lane_lt8 = lane < 8
    perms = [(lane ^ 1)[:, None], (lane ^ 2)[:, None], (lane ^ 4)[:, None],
             (lane ^ 8)[:, None]]
    gdn = lax.GatherDimensionNumbers(offset_dims=(), collapsed_slice_dims=(0,),
                                     start_index_map=(0,))
    shuf = lambda t, p: lax.gather(
        t, p, gdn, (1,), mode=lax.GatherScatterMode.PROMISE_IN_BOUNDS)

    @pl.loop(0, NCH)
    def _chunk(ci):
        base = wid * EPT + ci * W
        pltpu.sync_copy(src_hbm.at[pl.ds(base, W)], srci)
        pltpu.sync_copy(dst_hbm.at[pl.ds(base, W)], dsti)
        pltpu.sync_copy(ea_hbm.at[pl.ds(base, W), :], eabuf)
        pltpu.sync_copy(kv_hbm.at[srci], kvbuf)
        pltpu.sync_copy(q_hbm.at[dsti], qbuf)

        we0 = [wepbuf[0, pl.ds(h * 16, 16)] for h in range(H)]
        we1 = [wepbuf[1, pl.ds(h * 16, 16)] for h in range(H)]
        be = [wepbuf[2, pl.ds(h * 16, 16)] for h in range(H)]

        @plsc.parallel_loop(0, W)
        def _edge(w):
            wv = jnp.full((16,), w, jnp.int32)
            ea0 = plsc.load_gather(eabuf, [wv, zeros16i])
            ea1 = plsc.load_gather(eabuf, [wv, ones16i])
            es, ts, vse = [], [], []
            for h in range(H):
                es.append(ea0 * we0[h] + ea1 * we1[h] + be[h])
            for h in range(H):
                sl = pl.ds(h * 16, 16)
                ts.append(qbuf[w, sl] * (kvbuf[w, sl] + es[h]))
                vse.append(kvbuf[w, pl.ds(128 + h * 16, 16)] + es[h])
            for p in perms:
                ts = [t + shuf(t, p) for t in ts]
            exs = [jnp.exp(jnp.minimum(t, 60.0)) for t in ts]
            for h in range(H):
                msgbuf[w, pl.ds(h * 16, 16)] = vse[h] * exs[h]
            tail = jnp.zeros((16,), jnp.float32)
            for h in range(H):
                tail = jnp.where(lane == h, exs[h], tail)
            plsc.store_scatter(denbuf, [wv, lane], tail, mask=lane_lt8)

        pltpu.sync_copy(msgbuf, accm.at[dsti], add=True)
        pltpu.sync_copy(denbuf, accd.at[dsti], add=True)

    plsc.subcore_barrier()
    pltpu.sync_copy(accm.at[pl.ds(s * RPT, RPT), :],
                    outm_hbm.at[c, pl.ds(s * RPT, RPT), :])
    pltpu.sync_copy(accd.at[pl.ds(s * RPT, RPT), :],
                    outd_hbm.at[c, pl.ds(s * RPT, RPT), :])


def _edge_pass(q, kv, src, dst, ea, wepack, zeros, zerosd):
    mesh = plsc.VectorSubcoreMesh(core_axis_name="c", subcore_axis_name="s")
    k = functools.partial(
        pl.kernel,
        mesh=mesh,
        compiler_params=pltpu.CompilerParams(use_tc_tiling_on_sc=False,
                                             needs_layout_passes=False),
        out_type=[
            jax.ShapeDtypeStruct((2, N, HID), jnp.float32),
            jax.ShapeDtypeStruct((2, N, H), jnp.float32),
        ],
        scratch_types=[
            pltpu.VMEM((W,), jnp.int32),
            pltpu.VMEM((W,), jnp.int32),
            pltpu.VMEM((W, HID), jnp.float32),
            pltpu.VMEM((W, 2 * HID), jnp.float32),
            pltpu.VMEM((W, 2), jnp.float32),
            pltpu.VMEM((W, HID), jnp.float32),
            pltpu.VMEM((W, H), jnp.float32),
            pltpu.VMEM((3, HID), jnp.float32),
            pltpu.VMEM_SHARED((N, HID), jnp.float32),
            pltpu.VMEM_SHARED((N, H), jnp.float32),
        ],
    )(_edge_body)
    return k(q, kv, src, dst, ea, wepack, zeros, zerosd)


def _ln(x, g, b):
    m = jnp.mean(x, axis=-1, keepdims=True)
    v = jnp.mean((x - m) ** 2, axis=-1, keepdims=True)
    return (x - m) / jnp.sqrt(v + 1e-5) * g + b


def _embed_body(x_ref, pe_ref, win_ref, bin_ref, g_ref, b_ref, wpe_ref,
                bpe_ref, o_ref):
    h = _ln(x_ref[...] @ win_ref[...] + bin_ref[...], g_ref[...], b_ref[...])
    o_ref[...] = h + pe_ref[...] @ wpe_ref[...] + bpe_ref[...]


def _embed(x, pe, win, bin_, g, b, wpe, bpe):
    full = lambda shp: pl.BlockSpec(shp, lambda i: (0,) * len(shp))
    return pl.pallas_call(
        _embed_body,
        grid=(NROWB,),
        in_specs=[
            pl.BlockSpec((ROWB, HID), lambda i: (i, 0)),
            pl.BlockSpec((ROWB, 8), lambda i: (i, 0)),
            full((HID, HID)), full((1, HID)), full((1, HID)), full((1, HID)),
            full((8, HID)), full((1, HID)),
        ],
        out_specs=pl.BlockSpec((ROWB, HID), lambda i: (i, 0)),
        out_shape=jax.ShapeDtypeStruct((N, HID), jnp.float32),
    )(x, pe, win, bin_, g, b, wpe, bpe)


def _qkv_body(h_ref, w_ref, b_ref, q_ref, kv_ref):
    y = h_ref[...] @ w_ref[...] + b_ref[...]
    q_ref[...] = y[:, :HID]
    kv_ref[...] = y[:, HID:]


def _qkv(h, wqkv, bqkv):
    full = lambda shp: pl.BlockSpec(shp, lambda i: (0,) * len(shp))
    return pl.pallas_call(
        _qkv_body,
        grid=(NROWB,),
        in_specs=[
            pl.BlockSpec((ROWB, HID), lambda i: (i, 0)),
            full((HID, 3 * HID)), full((1, 3 * HID)),
        ],
        out_specs=[
            pl.BlockSpec((ROWB, HID), lambda i: (i, 0)),
            pl.BlockSpec((ROWB, 2 * HID), lambda i: (i, 0)),
        ],
        out_shape=[
            jax.ShapeDtypeStruct((N, HID), jnp.float32),
            jax.ShapeDtypeStruct((N, 2 * HID), jnp.float32),
        ],
    )(h, wqkv, bqkv)


def _post_body(accm_ref, accd_ref, h_ref, e8_ref, ws_ref, bs_ref, g_ref,
               b_ref, w1_ref, b1_ref, w2_ref, b2_ref, o_ref):
    a = accm_ref[0] + accm_ref[1]
    d = accd_ref[0] + accd_ref[1]
    denx = d @ e8_ref[...]
    attn = a / (denx + 1e-16)
    h = h_ref[...]
    h2 = attn + h @ ws_ref[...] + bs_ref[...]
    hh = _ln(h + h2, g_ref[...], b_ref[...])
    f = jnp.maximum(hh @ w1_ref[...] + b1_ref[...], 0.0) @ w2_ref[...] \
        + b2_ref[...]
    o_ref[...] = _ln(hh + f, g_ref[...], b_ref[...])


def _post(accm, accd, h, e8, ws, bs, g, b, w1, b1, w2, b2):
    full = lambda shp: pl.BlockSpec(shp, lambda i: (0,) * len(shp))
    return pl.pallas_call(
        _post_body,
        grid=(NROWB,),
        in_specs=[
            pl.BlockSpec((2, ROWB, HID), lambda i: (0, i, 0)),
            pl.BlockSpec((2, ROWB, H), lambda i: (0, i, 0)),
            pl.BlockSpec((ROWB, HID), lambda i: (i, 0)),
            full((H, HID)), full((HID, HID)), full((1, HID)),
            full((1, HID)), full((1, HID)),
            full((HID, 4 * HID)), full((1, 4 * HID)),
            full((4 * HID, HID)), full((1, HID)),
        ],
        out_specs=pl.BlockSpec((ROWB, HID), lambda i: (i, 0)),
        out_shape=jax.ShapeDtypeStruct((N, HID), jnp.float32),
    )(accm, accd, h, e8, ws, bs, g, b, w1, b1, w2, b2)


def _pool_body(h_ref, bt_ref, wt1_ref, bt1_ref, wt2_ref, bt2_ref, wg1_ref,
               bg1_ref, wg2_ref, bg2_ref, t_ref, g_ref, pacc, cacc):
    i = pl.program_id(0)

    @pl.when(i == 0)
    def _():
        pacc[...] = jnp.zeros((B, HID), jnp.float32)
        cacc[...] = jnp.zeros((B, HID), jnp.float32)

    bb = jnp.broadcast_to(bt_ref[0], (B, ROWB))
    ids = lax.broadcasted_iota(jnp.int32, (B, ROWB), 0).astype(jnp.float32)
    oh = (ids == bb).astype(jnp.float32)
    pacc[...] += oh @ h_ref[...]
    cacc[...] += jnp.broadcast_to(jnp.sum(oh, axis=1, keepdims=True), (B, HID))

    @pl.when(i == NROWB - 1)
    def _():
        pooled = pacc[...] / jnp.maximum(cacc[...], 1.0)
        z = jnp.maximum(pooled @ wt1_ref[...] + bt1_ref[...], 0.0)
        tv = z @ wt2_ref[...] + bt2_ref[...]
        t_ref[...] = jnp.broadcast_to(tv[:, :1], (B, HID))
        zg = jnp.maximum(pooled @ wg1_ref[...] + bg1_ref[...], 0.0)
        gv = jax.nn.sigmoid(zg @ wg2_ref[...] + bg2_ref[...]) * 2.0
        g_ref[...] = jnp.broadcast_to(gv[:, :1], (B, HID))


def _pool(h, batchf, wt1, bt1, wt2, bt2, wg1, bg1, wg2, bg2):
    full = lambda shp: pl.BlockSpec(shp, lambda i: (0,) * len(shp))
    out = pl.pallas_call(
        _pool_body,
        grid=(NROWB,),
        in_specs=[
            pl.BlockSpec((ROWB, HID), lambda i: (i, 0)),
            pl.BlockSpec((1, 1, ROWB), lambda i: (i, 0, 0)),
            full((HID, 64)), full((1, 64)), full((64, HID)), full((1, HID)),
            full((HID, 64)), full((1, 64)), full((64, HID)), full((1, HID)),
        ],
        out_specs=[
            pl.BlockSpec((B, HID), lambda i: (0, 0)),
            pl.BlockSpec((B, HID), lambda i: (0, 0)),
        ],
        out_shape=[
            jax.ShapeDtypeStruct((B, HID), jnp.float32),
            jax.ShapeDtypeStruct((B, HID), jnp.float32),
        ],
        scratch_shapes=[
            pltpu.VMEM((B, HID), jnp.float32),
            pltpu.VMEM((B, HID), jnp.float32),
        ],
    )(h, batchf, wt1, bt1, wt2, bt2, wg1, bg1, wg2, bg2)
    return out


_E8 = np.kron(np.eye(H, dtype=np.float32), np.ones((1, C), np.float32))


def kernel(x, edge_index, edge_attr, batch, pe, params):
    with jax.default_matmul_precision("highest"):
        return _kernel_impl(x, edge_index, edge_attr, batch, pe, params)


def _kernel_impl(x, edge_index, edge_attr, batch, pe, params):
    p = params
    src = edge_index[0].astype(jnp.int32)
    dst = edge_index[1].astype(jnp.int32)
    ea = edge_attr.astype(jnp.float32)
    e8 = jnp.asarray(_E8)
    zeros = jnp.zeros((RPT, HID), jnp.float32)
    zerosd = jnp.zeros((RPT, H), jnp.float32)
    r = lambda v: v.reshape(1, -1)

    h = _embed(x, pe, p['Win'], r(p['bin']), r(p['g_in']), r(p['b_in']),
               p['Wpe'], r(p['bpe']))

    for lp in p['layers']:
        wqkv = jnp.concatenate([lp['Wq'] * 0.25, lp['Wk'], lp['Wv']], axis=1)
        bqkv = jnp.concatenate([lp['bq'] * 0.25, lp['bk'],
                                lp['bv']]).reshape(1, -1)
        wepack = jnp.concatenate([lp['We'], lp['be'].reshape(1, -1)], axis=0)
        q, kv = _qkv(h, wqkv, bqkv)
        accm, accd = _edge_pass(q, kv, src, dst, ea, wepack, zeros, zerosd)
        h = _post(accm, accd, h, e8, lp['Ws'], r(lp['bs']), r(lp['ln_g']),
                  r(lp['ln_b']), lp['W1'], r(lp['b1']), lp['W2'], r(lp['b2']))

    batchf = batch.astype(jnp.float32).reshape(NROWB, 1, ROWB)
    pad2 = lambda w: jnp.pad(w, ((0, 0), (0, HID - w.shape[1])))
    padb = lambda v: jnp.pad(v.reshape(1, -1), ((0, 0), (0, HID - v.shape[0])))
    t2, g2 = _pool(h, batchf, p['Wt1'], r(p['bt1']), pad2(p['Wt2']),
                   padb(p['bt2']), p['Wg1'], r(p['bg1']), pad2(p['Wg2']),
                   padb(p['bg2']))
    return (t2[:, 0], g2[:, 0])
